# Initial kernel scaffold; baseline (speedup 1.0000x reference)
import jax
import jax.numpy as jnp
from jax.experimental import pallas as pl


def kernel(x, image):
    raise NotImplementedError("write your pallas kernel here")


# speedup vs baseline: 1.0088x; 1.0088x over previous
import jax
import jax.numpy as jnp
from jax.experimental import pallas as pl

_THETA_ALPHA = 16.0
_THETA_BETA = 0.5
_ROW_T = 512
_COL_T = 4096
_D_PAD = 8
_C_PAD = 128


def _pairwise_body(frow_ref, fb_ref, fsq_ref, xv_ref, o_ref):
    n = fb_ref.shape[0]
    fc = frow_ref[...]
    fcsq = jnp.sum(fc * fc, axis=1, keepdims=True)
    fcb = fc.astype(jnp.bfloat16)
    acc0 = jnp.zeros((fc.shape[0], _C_PAD), jnp.float32)

    def body(j, acc):
        fjb = fb_ref[pl.ds(j * _COL_T, _COL_T), :]
        fjsq = fsq_ref[:, pl.ds(j * _COL_T, _COL_T)]
        xj = xv_ref[pl.ds(j * _COL_T, _COL_T), :]
        s = jax.lax.dot_general(
            fcb, fjb, (((1,), (1,)), ((), ())),
            preferred_element_type=jnp.float32)
        d2 = (fcsq + fjsq) - 2.0 * s
        k = jnp.exp(-0.5 * jnp.maximum(d2, 0.0)).astype(jnp.bfloat16)
        return acc + jax.lax.dot_general(
            k, xj, (((1,), (0,)), ((), ())),
            preferred_element_type=jnp.float32)

    o_ref[...] = jax.lax.fori_loop(0, n // _COL_T, body, acc0)


def _gauss_filter_pallas(x_flat, f):
    n, c = x_flat.shape
    fp = jnp.zeros((n, _D_PAD), jnp.float32).at[:, :f.shape[1]].set(f)
    fsq = jnp.sum(f * f, axis=-1)[None, :]
    xv = jnp.zeros((n, _C_PAD), jnp.bfloat16).at[:, :c].set(
        x_flat.astype(jnp.bfloat16))

    out = pl.pallas_call(
        _pairwise_body,
        grid=(n // _ROW_T,),
        in_specs=[
            pl.BlockSpec((_ROW_T, _D_PAD), lambda i: (i, 0)),
            pl.BlockSpec((n, _D_PAD), lambda i: (0, 0)),
            pl.BlockSpec((1, n), lambda i: (0, 0)),
            pl.BlockSpec((n, _C_PAD), lambda i: (0, 0)),
        ],
        out_specs=pl.BlockSpec((_ROW_T, _C_PAD), lambda i: (i, 0)),
        out_shape=jax.ShapeDtypeStruct((n, _C_PAD), jnp.float32),
    )(fp, fp.astype(jnp.bfloat16), fsq, xv)
    return out[:, :c]


def kernel(x, image):
    bsz, c, h, w = x.shape
    n = h * w
    yy, xx = jnp.meshgrid(
        jnp.arange(h, dtype=jnp.float32),
        jnp.arange(w, dtype=jnp.float32),
        indexing="ij",
    )
    pos = jnp.stack([yy, xx], axis=-1).reshape(n, 2) / _THETA_ALPHA

    outs = []
    for bi in range(bsz):
        img_flat = image[bi].reshape(image.shape[1], n).T / _THETA_BETA
        f = jnp.concatenate([pos, img_flat], axis=1)
        x_flat = x[bi].reshape(c, n).T
        out = _gauss_filter_pallas(x_flat, f)
        outs.append(out.T.reshape(c, h, w))
    return jnp.stack(outs, axis=0)


# speedup vs baseline: 1.3816x; 1.3695x over previous
import jax
import jax.numpy as jnp
from jax.experimental import pallas as pl

_THETA_ALPHA = 16.0
_THETA_BETA = 0.5
_ROW_T = 512
_COL_T = 4096
_D_PAD = 16
_C_PAD = 128


def _pairwise_body(arow_ref, b_ref, xv_ref, o_ref):
    n = b_ref.shape[0]
    a = arow_ref[...]
    acc0 = jnp.zeros((a.shape[0], _C_PAD), jnp.float32)

    def body(j, acc):
        bj = b_ref[pl.ds(j * _COL_T, _COL_T), :]
        xj = xv_ref[pl.ds(j * _COL_T, _COL_T), :]
        e = jax.lax.dot_general(
            a, bj, (((1,), (1,)), ((), ())),
            preferred_element_type=jnp.float32)
        k = jnp.exp(jnp.minimum(e, 0.0)).astype(jnp.bfloat16)
        return acc + jax.lax.dot_general(
            k, xj, (((1,), (0,)), ((), ())),
            preferred_element_type=jnp.float32)

    o_ref[...] = jax.lax.fori_loop(0, n // _COL_T, body, acc0)


def _gauss_filter_pallas(x_flat, f):
    n, c = x_flat.shape
    d = f.shape[1]
    fb = f.astype(jnp.bfloat16)
    hc = -0.5 * jnp.sum(f * f, axis=-1, keepdims=True)
    h_hi = jax.lax.optimization_barrier(hc.astype(jnp.bfloat16))
    h_lo = (hc - h_hi.astype(jnp.float32)).astype(jnp.bfloat16)
    ones = jnp.ones((n, 1), jnp.bfloat16)
    pad = jnp.zeros((n, _D_PAD - d - 4), jnp.bfloat16)
    ab = jnp.concatenate([fb, h_hi, h_lo, ones, ones, pad], axis=1)
    bb = jnp.concatenate([fb, ones, ones, h_hi, h_lo, pad], axis=1)
    xv = jnp.zeros((n, _C_PAD), jnp.bfloat16).at[:, :c].set(
        x_flat.astype(jnp.bfloat16))

    out = pl.pallas_call(
        _pairwise_body,
        grid=(n // _ROW_T,),
        in_specs=[
            pl.BlockSpec((_ROW_T, _D_PAD), lambda i: (i, 0)),
            pl.BlockSpec((n, _D_PAD), lambda i: (0, 0)),
            pl.BlockSpec((n, _C_PAD), lambda i: (0, 0)),
        ],
        out_specs=pl.BlockSpec((_ROW_T, _C_PAD), lambda i: (i, 0)),
        out_shape=jax.ShapeDtypeStruct((n, _C_PAD), jnp.float32),
    )(ab, bb, xv)
    return out[:, :c]


def kernel(x, image):
    bsz, c, h, w = x.shape
    n = h * w
    yy, xx = jnp.meshgrid(
        jnp.arange(h, dtype=jnp.float32),
        jnp.arange(w, dtype=jnp.float32),
        indexing="ij",
    )
    pos = jnp.stack([yy, xx], axis=-1).reshape(n, 2) / _THETA_ALPHA

    outs = []
    for bi in range(bsz):
        img_flat = image[bi].reshape(image.shape[1], n).T / _THETA_BETA
        f = jnp.concatenate([pos, img_flat], axis=1)
        x_flat = x[bi].reshape(c, n).T
        out = _gauss_filter_pallas(x_flat, f)
        outs.append(out.T.reshape(c, h, w))
    return jnp.stack(outs, axis=0)


# speedup vs baseline: 1.6890x; 1.2226x over previous
import jax
import jax.numpy as jnp
from jax.experimental import pallas as pl

_THETA_ALPHA = 16.0
_THETA_BETA = 0.5
_ROW_T = 512
_COL_T = 4096
_D_PAD = 16
_C_PAD = 32


def _pairwise_body(at_ref, b_ref, xt_ref, o_ref):
    n = b_ref.shape[0]
    at = at_ref[...]
    acc0 = jnp.zeros((_C_PAD, at.shape[1]), jnp.float32)

    def body(j, acc):
        bj = b_ref[pl.ds(j * _COL_T, _COL_T), :]
        xtj = xt_ref[:, pl.ds(j * _COL_T, _COL_T)]
        et = jax.lax.dot_general(
            bj, at, (((1,), (0,)), ((), ())),
            preferred_element_type=jnp.float32)
        kt = jnp.exp(jnp.minimum(et, 0.0)).astype(jnp.bfloat16)
        return acc + jax.lax.dot_general(
            xtj, kt, (((1,), (0,)), ((), ())),
            preferred_element_type=jnp.float32)

    o_ref[...] = jax.lax.fori_loop(0, n // _COL_T, body, acc0)


def _gauss_filter_pallas(x_flat, f):
    n, c = x_flat.shape
    d = f.shape[1]
    fb = f.astype(jnp.bfloat16)
    hc = -0.5 * jnp.sum(f * f, axis=-1, keepdims=True)
    h_hi = jax.lax.optimization_barrier(hc.astype(jnp.bfloat16))
    h_lo = (hc - h_hi.astype(jnp.float32)).astype(jnp.bfloat16)
    ones = jnp.ones((n, 1), jnp.bfloat16)
    pad = jnp.zeros((n, _D_PAD - d - 4), jnp.bfloat16)
    at = jnp.concatenate([fb, h_hi, h_lo, ones, ones, pad], axis=1).T
    bb = jnp.concatenate([fb, ones, ones, h_hi, h_lo, pad], axis=1)
    xt = jnp.zeros((_C_PAD, n), jnp.bfloat16).at[:c, :].set(
        x_flat.T.astype(jnp.bfloat16))

    out_t = pl.pallas_call(
        _pairwise_body,
        grid=(n // _ROW_T,),
        in_specs=[
            pl.BlockSpec((_D_PAD, _ROW_T), lambda i: (0, i)),
            pl.BlockSpec((n, _D_PAD), lambda i: (0, 0)),
            pl.BlockSpec((_C_PAD, n), lambda i: (0, 0)),
        ],
        out_specs=pl.BlockSpec((_C_PAD, _ROW_T), lambda i: (0, i)),
        out_shape=jax.ShapeDtypeStruct((_C_PAD, n), jnp.float32),
    )(at, bb, xt)
    return out_t[:c, :]


def kernel(x, image):
    bsz, c, h, w = x.shape
    n = h * w
    yy, xx = jnp.meshgrid(
        jnp.arange(h, dtype=jnp.float32),
        jnp.arange(w, dtype=jnp.float32),
        indexing="ij",
    )
    pos = jnp.stack([yy, xx], axis=-1).reshape(n, 2) / _THETA_ALPHA

    outs = []
    for bi in range(bsz):
        img_flat = image[bi].reshape(image.shape[1], n).T / _THETA_BETA
        f = jnp.concatenate([pos, img_flat], axis=1)
        x_flat = x[bi].reshape(c, n).T
        out_t = _gauss_filter_pallas(x_flat, f)
        outs.append(out_t.reshape(c, h, w))
    return jnp.stack(outs, axis=0)


# speedup vs baseline: 2.0209x; 1.1965x over previous
import jax
import jax.numpy as jnp
from jax.experimental import pallas as pl

_THETA_ALPHA = 16.0
_THETA_BETA = 0.5
_T = 1024
_D_PAD = 16
_C_PAD = 32


def _pairwise_body(at_ref, b_ref, xt_ref, o_ref):
    n = b_ref.shape[0]
    nb = n // _T
    o_ref[...] = jnp.zeros_like(o_ref)

    def outer(i, carry):
        ati = at_ref[:, pl.ds(i * _T, _T)]
        xti = xt_ref[:, pl.ds(i * _T, _T)]
        bi = b_ref[pl.ds(i * _T, _T), :]

        ed = jax.lax.dot_general(
            bi, ati, (((1,), (0,)), ((), ())),
            preferred_element_type=jnp.float32)
        kd = jnp.exp(jnp.minimum(ed, 0.0)).astype(jnp.bfloat16)
        o_ref[:, pl.ds(i * _T, _T)] += jax.lax.dot_general(
            xti, kd, (((1,), (0,)), ((), ())),
            preferred_element_type=jnp.float32)

        def inner(j, carry2):
            bj = b_ref[pl.ds(j * _T, _T), :]
            xtj = xt_ref[:, pl.ds(j * _T, _T)]
            e = jax.lax.dot_general(
                bj, ati, (((1,), (0,)), ((), ())),
                preferred_element_type=jnp.float32)
            k = jnp.exp(jnp.minimum(e, 0.0)).astype(jnp.bfloat16)
            o_ref[:, pl.ds(i * _T, _T)] += jax.lax.dot_general(
                xtj, k, (((1,), (0,)), ((), ())),
                preferred_element_type=jnp.float32)
            o_ref[:, pl.ds(j * _T, _T)] += jax.lax.dot_general(
                xti, k, (((1,), (1,)), ((), ())),
                preferred_element_type=jnp.float32)
            return carry2

        return jax.lax.fori_loop(i + 1, nb, inner, carry)

    jax.lax.fori_loop(0, nb, outer, 0)


def _gauss_filter_pallas(x_flat, f):
    n, c = x_flat.shape
    d = f.shape[1]
    fb = f.astype(jnp.bfloat16)
    hc = -0.5 * jnp.sum(f * f, axis=-1, keepdims=True)
    h_hi = jax.lax.optimization_barrier(hc.astype(jnp.bfloat16))
    h_lo = (hc - h_hi.astype(jnp.float32)).astype(jnp.bfloat16)
    ones = jnp.ones((n, 1), jnp.bfloat16)
    pad = jnp.zeros((n, _D_PAD - d - 4), jnp.bfloat16)
    at = jnp.concatenate([fb, h_hi, h_lo, ones, ones, pad], axis=1).T
    bb = jnp.concatenate([fb, ones, ones, h_hi, h_lo, pad], axis=1)
    xt = jnp.zeros((_C_PAD, n), jnp.bfloat16).at[:c, :].set(
        x_flat.T.astype(jnp.bfloat16))

    out_t = pl.pallas_call(
        _pairwise_body,
        grid=(1,),
        in_specs=[
            pl.BlockSpec((_D_PAD, n), lambda i: (0, 0)),
            pl.BlockSpec((n, _D_PAD), lambda i: (0, 0)),
            pl.BlockSpec((_C_PAD, n), lambda i: (0, 0)),
        ],
        out_specs=pl.BlockSpec((_C_PAD, n), lambda i: (0, 0)),
        out_shape=jax.ShapeDtypeStruct((_C_PAD, n), jnp.float32),
    )(at, bb, xt)
    return out_t[:c, :]


def kernel(x, image):
    bsz, c, h, w = x.shape
    n = h * w
    yy, xx = jnp.meshgrid(
        jnp.arange(h, dtype=jnp.float32),
        jnp.arange(w, dtype=jnp.float32),
        indexing="ij",
    )
    pos = jnp.stack([yy, xx], axis=-1).reshape(n, 2) / _THETA_ALPHA

    outs = []
    for bi in range(bsz):
        img_flat = image[bi].reshape(image.shape[1], n).T / _THETA_BETA
        f = jnp.concatenate([pos, img_flat], axis=1)
        x_flat = x[bi].reshape(c, n).T
        out_t = _gauss_filter_pallas(x_flat, f)
        outs.append(out_t.reshape(c, h, w))
    return jnp.stack(outs, axis=0)


# speedup vs baseline: 2.1058x; 1.0420x over previous
import jax
import jax.numpy as jnp
from jax.experimental import pallas as pl

_THETA_ALPHA = 16.0
_THETA_BETA = 0.5
_T = 1024
_D_PAD = 16
_C_PAD = 32


def _pairwise_body(at_ref, b_ref, xt_ref, o_ref):
    n = b_ref.shape[0]
    nb = n // _T
    o_ref[...] = jnp.zeros_like(o_ref)

    def tile_k(i, j):
        ati = at_ref[:, pl.ds(i * _T, _T)]
        bj = b_ref[pl.ds(j * _T, _T), :]
        e = jax.lax.dot_general(
            bj, ati, (((1,), (0,)), ((), ())),
            preferred_element_type=jnp.float32)
        return jnp.exp(jnp.minimum(e, 0.0)).astype(jnp.bfloat16)

    def accum(i, j, k):
        xtj = xt_ref[:, pl.ds(j * _T, _T)]
        o_ref[:, pl.ds(i * _T, _T)] += jax.lax.dot_general(
            xtj, k, (((1,), (0,)), ((), ())),
            preferred_element_type=jnp.float32)
        xti = xt_ref[:, pl.ds(i * _T, _T)]
        o_ref[:, pl.ds(j * _T, _T)] += jax.lax.dot_general(
            xti, k, (((1,), (1,)), ((), ())),
            preferred_element_type=jnp.float32)

    def pair2(i1, j1, i2, j2):
        k1 = tile_k(i1, j1)
        k2 = tile_k(i2, j2)
        accum(i1, j1, k1)
        accum(i2, j2, k2)

    def diag2(t, carry):
        i1, i2 = 2 * t, 2 * t + 1
        k1 = tile_k(i1, i1)
        k2 = tile_k(i2, i2)
        xt1 = xt_ref[:, pl.ds(i1 * _T, _T)]
        o_ref[:, pl.ds(i1 * _T, _T)] += jax.lax.dot_general(
            xt1, k1, (((1,), (0,)), ((), ())),
            preferred_element_type=jnp.float32)
        xt2 = xt_ref[:, pl.ds(i2 * _T, _T)]
        o_ref[:, pl.ds(i2 * _T, _T)] += jax.lax.dot_general(
            xt2, k2, (((1,), (0,)), ((), ())),
            preferred_element_type=jnp.float32)
        return carry

    jax.lax.fori_loop(0, nb // 2, diag2, 0)

    for d in range(1, nb // 2):
        def offd(t, carry, d=d):
            i1, i2 = 2 * t, 2 * t + 1
            pair2(i1, (i1 + d) % nb, i2, (i2 + d) % nb)
            return carry

        jax.lax.fori_loop(0, nb // 2, offd, 0)

    def half(t, carry):
        i1, i2 = 2 * t, 2 * t + 1
        pair2(i1, i1 + nb // 2, i2, i2 + nb // 2)
        return carry

    jax.lax.fori_loop(0, nb // 4, half, 0)


def _gauss_filter_pallas(x_flat, f):
    n, c = x_flat.shape
    d = f.shape[1]
    fb = f.astype(jnp.bfloat16)
    hc = -0.5 * jnp.sum(f * f, axis=-1, keepdims=True)
    h_hi = jax.lax.optimization_barrier(hc.astype(jnp.bfloat16))
    h_lo = (hc - h_hi.astype(jnp.float32)).astype(jnp.bfloat16)
    ones = jnp.ones((n, 1), jnp.bfloat16)
    pad = jnp.zeros((n, _D_PAD - d - 4), jnp.bfloat16)
    at = jnp.concatenate([fb, h_hi, h_lo, ones, ones, pad], axis=1).T
    bb = jnp.concatenate([fb, ones, ones, h_hi, h_lo, pad], axis=1)
    xt = jnp.zeros((_C_PAD, n), jnp.bfloat16).at[:c, :].set(
        x_flat.T.astype(jnp.bfloat16))

    out_t = pl.pallas_call(
        _pairwise_body,
        grid=(1,),
        in_specs=[
            pl.BlockSpec((_D_PAD, n), lambda i: (0, 0)),
            pl.BlockSpec((n, _D_PAD), lambda i: (0, 0)),
            pl.BlockSpec((_C_PAD, n), lambda i: (0, 0)),
        ],
        out_specs=pl.BlockSpec((_C_PAD, n), lambda i: (0, 0)),
        out_shape=jax.ShapeDtypeStruct((_C_PAD, n), jnp.float32),
    )(at, bb, xt)
    return out_t[:c, :]


def kernel(x, image):
    bsz, c, h, w = x.shape
    n = h * w
    yy, xx = jnp.meshgrid(
        jnp.arange(h, dtype=jnp.float32),
        jnp.arange(w, dtype=jnp.float32),
        indexing="ij",
    )
    pos = jnp.stack([yy, xx], axis=-1).reshape(n, 2) / _THETA_ALPHA

    outs = []
    for bi in range(bsz):
        img_flat = image[bi].reshape(image.shape[1], n).T / _THETA_BETA
        f = jnp.concatenate([pos, img_flat], axis=1)
        x_flat = x[bi].reshape(c, n).T
        out_t = _gauss_filter_pallas(x_flat, f)
        outs.append(out_t.reshape(c, h, w))
    return jnp.stack(outs, axis=0)
